# baseline (speedup 1.0000x reference)
import jax
import jax.numpy as jnp
from jax import lax
from jax.experimental import pallas as pl
from jax.experimental.pallas import tpu as pltpu
from jax.experimental.pallas import tpu_sc as plsc

_N_POSES = 4096
_MAX_BLOCKS = 100
_N_BLOCK_TYPES = 1000

_NUM_CORES = 1
_NUM_SUBCORES = 16
_NW = _NUM_CORES * _NUM_SUBCORES
_PPW = _N_POSES // _NW
_LANES = 16
_GROUPS = _PPW // _LANES
_HALF = _GROUPS // 2
_CHUNK = _PPW * _MAX_BLOCKS
_WPAD = _N_BLOCK_TYPES + _LANES


_QUARTERS = 4
_GPQ = _GROUPS // _QUARTERS


def _sc_body(bt_hbm, w_hbm, out_hbm, w_v, bt_v, out_v, *sems):
    wid = lax.axis_index("s") * _NUM_CORES + lax.axis_index("c")
    base = wid * _CHUNK
    qlen = _CHUNK // _QUARTERS
    wcp = pltpu.async_copy(
        w_hbm, w_v.at[pl.ds(0, _N_BLOCK_TYPES)], sems[_QUARTERS])
    cps = [
        pltpu.async_copy(
            bt_hbm.at[pl.ds(base + q * qlen, qlen)],
            bt_v.at[pl.ds(q * qlen, qlen)],
            sems[q],
        )
        for q in range(_QUARTERS)
    ]
    w_v[pl.ds(_N_BLOCK_TYPES, _LANES)] = jnp.zeros((_LANES,), jnp.float32)

    lanes = lax.iota(jnp.int32, _LANES)
    row_offs = [(lanes + g * _LANES) * _MAX_BLOCKS for g in range(_GROUPS)]
    sentinel = jnp.full((_LANES,), _N_BLOCK_TYPES, jnp.int32)

    def make_bstep(g0):
        def bstep(b, accs):
            new = []
            for g in range(g0, g0 + _GPQ):
                idx = plsc.load_gather(bt_v, [row_offs[g] + b])
                safe = jnp.where(idx < 0, sentinel, idx)
                new.append(accs[g - g0] + plsc.load_gather(w_v, [safe]))
            return tuple(new)
        return bstep

    zeros = tuple(jnp.zeros((_LANES,), jnp.float32) for _ in range(_GPQ))
    wcp.wait()
    for q in range(_QUARTERS):
        cps[q].wait()
        accs = lax.fori_loop(
            0, _MAX_BLOCKS, make_bstep(q * _GPQ), zeros, unroll=5)
        for g in range(_GPQ):
            out_v[pl.ds((q * _GPQ + g) * _LANES, _LANES)] = accs[g]

    pltpu.sync_copy(out_v, out_hbm.at[pl.ds(wid * _PPW, _PPW)])


@jax.jit
def _score(pose_stack_block_types, ref_weights):
    mesh = plsc.VectorSubcoreMesh(
        core_axis_name="c", subcore_axis_name="s", num_cores=_NUM_CORES
    )
    run = pl.kernel(
        _sc_body,
        out_type=jax.ShapeDtypeStruct((_N_POSES,), jnp.float32),
        mesh=mesh,
        compiler_params=pltpu.CompilerParams(needs_layout_passes=False),
        scratch_types=[
            pltpu.VMEM((_WPAD,), jnp.float32),
            pltpu.VMEM((_CHUNK,), jnp.int32),
            pltpu.VMEM((_PPW,), jnp.float32),
        ] + [pltpu.SemaphoreType.DMA] * (_QUARTERS + 1),
    )
    return run(pose_stack_block_types.reshape(-1), ref_weights)


def kernel(coords, pose_stack_block_types, ref_weights):
    del coords
    out = _score(pose_stack_block_types, ref_weights)
    return out.reshape(1, _N_POSES)


# speedup vs baseline: 1.0125x; 1.0125x over previous
import jax
import jax.numpy as jnp
from jax import lax
from jax.experimental import pallas as pl
from jax.experimental.pallas import tpu as pltpu
from jax.experimental.pallas import tpu_sc as plsc

_N_POSES = 4096
_MAX_BLOCKS = 100
_N_BLOCK_TYPES = 1000

_NUM_CORES = 1
_NUM_SUBCORES = 16
_NW = _NUM_CORES * _NUM_SUBCORES
_PPW = _N_POSES // _NW
_LANES = 16
_GROUPS = _PPW // _LANES
_HALF = _GROUPS // 2
_CHUNK = _PPW * _MAX_BLOCKS
_WPAD = _N_BLOCK_TYPES + _LANES


_QUARTERS = 4
_GPQ = _GROUPS // _QUARTERS


def _sc_body(bt_hbm, w_hbm, out_hbm, w_v, bt_v, out_v, *sems):
    wid = lax.axis_index("s") * _NUM_CORES + lax.axis_index("c")
    base = wid * _CHUNK
    qlen = _CHUNK // _QUARTERS
    wcp = pltpu.async_copy(
        w_hbm, w_v.at[pl.ds(0, _N_BLOCK_TYPES)], sems[_QUARTERS])
    cps = [
        pltpu.async_copy(
            bt_hbm.at[pl.ds(base + q * qlen, qlen)],
            bt_v.at[pl.ds(q * qlen, qlen)],
            sems[q],
        )
        for q in range(_QUARTERS)
    ]
    w_v[pl.ds(_N_BLOCK_TYPES, _LANES)] = jnp.zeros((_LANES,), jnp.float32)

    lanes = lax.iota(jnp.int32, _LANES)
    row_offs = [(lanes + g * _LANES) * _MAX_BLOCKS for g in range(_GROUPS)]
    sentinel = jnp.full((_LANES,), _N_BLOCK_TYPES, jnp.int32)

    def make_bstep(g0):
        def bstep(b, accs):
            new = []
            for g in range(g0, g0 + _GPQ):
                idx = plsc.load_gather(bt_v, [row_offs[g] + b])
                safe = jnp.where(idx < 0, sentinel, idx)
                new.append(accs[g - g0] + plsc.load_gather(w_v, [safe]))
            return tuple(new)
        return bstep

    zeros = tuple(jnp.zeros((_LANES,), jnp.float32) for _ in range(_GPQ))
    wcp.wait()
    for q in range(_QUARTERS):
        cps[q].wait()
        accs = lax.fori_loop(
            0, _MAX_BLOCKS, make_bstep(q * _GPQ), zeros, unroll=4)
        for g in range(_GPQ):
            out_v[pl.ds((q * _GPQ + g) * _LANES, _LANES)] = accs[g]

    pltpu.sync_copy(out_v, out_hbm.at[pl.ds(wid * _PPW, _PPW)])


@jax.jit
def _score(pose_stack_block_types, ref_weights):
    mesh = plsc.VectorSubcoreMesh(
        core_axis_name="c", subcore_axis_name="s", num_cores=_NUM_CORES
    )
    run = pl.kernel(
        _sc_body,
        out_type=jax.ShapeDtypeStruct((_N_POSES,), jnp.float32),
        mesh=mesh,
        compiler_params=pltpu.CompilerParams(needs_layout_passes=False),
        scratch_types=[
            pltpu.VMEM((_WPAD,), jnp.float32),
            pltpu.VMEM((_CHUNK,), jnp.int32),
            pltpu.VMEM((_PPW,), jnp.float32),
        ] + [pltpu.SemaphoreType.DMA] * (_QUARTERS + 1),
    )
    return run(pose_stack_block_types.reshape(-1), ref_weights)


def kernel(coords, pose_stack_block_types, ref_weights):
    del coords
    out = _score(pose_stack_block_types, ref_weights)
    return out.reshape(1, _N_POSES)
